# Initial kernel scaffold; baseline (speedup 1.0000x reference)
import jax
import jax.numpy as jnp
from jax.experimental import pallas as pl


def kernel(h, context, edge_index, Wc, bc, gate_W, gate_b):
    raise NotImplementedError("write your pallas kernel here")


# speedup vs baseline: 4.5398x; 4.5398x over previous
import functools

import jax
import jax.numpy as jnp
from jax import lax
from jax.experimental import pallas as pl
from jax.experimental.pallas import tpu as pltpu
from jax.experimental.pallas import tpu_sc as plsc

N = 10000
E = 160000
D = 256
HALF = 128
L = 2
EPS = 0.3

NC = 2
NS = 16
LANE = 16

EB = 128
EPT = 10240
NB = EPT // EB
NBH = NB // NC
EPAD = NS * EPT

STRIPE = 640
RC = 40

M_H = 10240
NPAD = N + 8

HS_H = 2 * N + 1000
R_TC = 1000

_MESH = dict(core_axis_name="c", subcore_axis_name="s", num_cores=NC,
             num_subcores=NS)
_SC_PARAMS = dict(compiler_params=pltpu.CompilerParams(
    needs_layout_passes=False))


def _zero16():
  return jnp.zeros((LANE,), jnp.float32)


def _lane_splat(vec, lane):
  idx = jnp.full((LANE,), lane, dtype=jnp.int32)
  return jnp.take_along_axis(vec, idx, axis=0,
                             mode=lax.GatherScatterMode.PROMISE_IN_BOUNDS)


def _deg_body(dst_hbm, out_hbm, dst_v, ones_v, zer_v, deg_sh):
  c = lax.axis_index("c")
  s = lax.axis_index("s")
  pltpu.sync_copy(dst_hbm.at[s], dst_v)

  for k in range(EB // LANE):
    ones_v[pl.ds(k * LANE, LANE)] = jnp.ones((LANE,), jnp.float32)

  def zb(i, carry):
    zer_v[pl.ds(i * LANE, LANE)] = _zero16()
    return carry
  lax.fori_loop(0, (M_H // NS) // LANE, zb, 0)
  pltpu.sync_copy(zer_v, deg_sh.at[pl.ds(s * (M_H // NS), M_H // NS)])
  plsc.subcore_barrier()

  def bb(b, carry):
    pltpu.sync_copy(ones_v, deg_sh.at[dst_v.at[c * NBH + b]], add=True)
    return carry
  lax.fori_loop(0, NBH, bb, 0)
  plsc.subcore_barrier()

  pltpu.sync_copy(deg_sh.at[pl.ds(s * (M_H // NS), M_H // NS)],
                  out_hbm.at[c].at[pl.ds(s * (M_H // NS), M_H // NS)])


@functools.cache
def _deg_kernel():
  mesh = plsc.VectorSubcoreMesh(**_MESH)
  return pl.kernel(
      _deg_body,
      out_type=jax.ShapeDtypeStruct((NC, M_H), jnp.float32),
      mesh=mesh,
      scratch_types=[
          pltpu.VMEM((NB, EB), jnp.int32),
          pltpu.VMEM((EB,), jnp.float32),
          pltpu.VMEM((M_H // NS,), jnp.float32),
          pltpu.VMEM_SHARED((M_H,), jnp.float32),
      ],
      **_SC_PARAMS,
  )


def _gate_body(src_hbm, dst_hbm, u_hbm, v_hbm, d_hbm, out_hbm,
               u_v, v_v, d_v, src_v, dst_v, e_v):
  c = lax.axis_index("c")
  s = lax.axis_index("s")
  pltpu.sync_copy(u_hbm, u_v)
  pltpu.sync_copy(v_hbm, v_v)
  pltpu.sync_copy(d_hbm, d_v)
  pltpu.sync_copy(src_hbm.at[s, pl.ds(c * NBH, NBH)], src_v)
  pltpu.sync_copy(dst_hbm.at[s, pl.ds(c * NBH, NBH)], dst_v)

  def bb(b, carry):
    def gb(g, carry2):
      off = pl.multiple_of(g * LANE, LANE)
      dst16 = dst_v[b, pl.ds(off, LANE)]
      src16 = src_v[b, pl.ds(off, LANE)]
      ug = plsc.load_gather(u_v, [dst16])
      vg = plsc.load_gather(v_v, [src16])
      dd = plsc.load_gather(d_v, [dst16])
      dsrc = plsc.load_gather(d_v, [src16])
      t = ug + vg
      ex = jnp.exp(t + t)
      gate = 1.0 - 2.0 / (ex + 1.0)
      e_v[b, pl.ds(off, LANE)] = gate * dd * dsrc
      return carry2
    return lax.fori_loop(0, EB // LANE, gb, carry)
  lax.fori_loop(0, NBH, bb, 0)

  pltpu.sync_copy(e_v, out_hbm.at[s, pl.ds(c * NBH, NBH)])


@functools.cache
def _gate_kernel():
  mesh = plsc.VectorSubcoreMesh(**_MESH)
  return pl.kernel(
      _gate_body,
      out_type=jax.ShapeDtypeStruct((NS, NB, EB), jnp.float32),
      mesh=mesh,
      scratch_types=[
          pltpu.VMEM((NPAD,), jnp.float32),
          pltpu.VMEM((NPAD,), jnp.float32),
          pltpu.VMEM((NPAD,), jnp.float32),
          pltpu.VMEM((NBH, EB), jnp.int32),
          pltpu.VMEM((NBH, EB), jnp.int32),
          pltpu.VMEM((NBH, EB), jnp.float32),
      ],
      **_SC_PARAMS,
  )


def _agg_body(hs_hbm, attn_hbm, src_hbm, dst_hbm, e_hbm, out_hbm,
              src_v, srco_v, dst_v, e_v, rowbuf, hbuf, abuf, mbuf, m_sh):
  c = lax.axis_index("c")
  s = lax.axis_index("s")

  def zb(i, carry):
    for k in range(HALF // LANE):
      hbuf[i, pl.ds(k * LANE, LANE)] = _zero16()
    return carry
  lax.fori_loop(0, RC, zb, 0)
  for k in range(STRIPE // RC):
    pltpu.sync_copy(hbuf, m_sh.at[pl.ds(s * STRIPE + k * RC, RC)])
  plsc.subcore_barrier()

  coff = c * N

  def batch_body(b, carry):
    base = (s * NB + b) * EB
    pltpu.sync_copy(src_hbm.at[pl.ds(base, EB)], src_v)
    pltpu.sync_copy(dst_hbm.at[pl.ds(base, EB)], dst_v)
    pltpu.sync_copy(e_hbm.at[pl.ds(base, EB)], e_v)
    for g in range(EB // LANE):
      sl = pl.ds(g * LANE, LANE)
      srco_v[sl] = src_v[sl] + coff
    pltpu.sync_copy(hs_hbm.at[srco_v], rowbuf)

    def grp_body(g, carry2):
      off = pl.multiple_of(g * LANE, LANE)
      e16 = e_v[pl.ds(off, LANE)]
      for t16 in range(LANE):
        e_bc = _lane_splat(e16, t16)
        j = off + t16
        for k in range(HALF // LANE):
          sl = pl.ds(k * LANE, LANE)
          rowbuf[j, sl] = rowbuf[j, sl] * e_bc
      return carry2
    lax.fori_loop(0, EB // LANE, grp_body, 0)

    pltpu.sync_copy(rowbuf, m_sh.at[dst_v], add=True)
    return carry
  lax.fori_loop(0, NB, batch_body, 0)
  plsc.subcore_barrier()

  row_base = coff + s * STRIPE
  mrow_base = s * STRIPE
  nch = lax.select(s == NS - 1, (N - (NS - 1) * STRIPE) // RC, STRIPE // RC)
  def ep_body(k2, carry):
    r0 = row_base + k2 * RC
    m0 = mrow_base + k2 * RC
    pltpu.sync_copy(hs_hbm.at[pl.ds(r0, RC)], hbuf)
    pltpu.sync_copy(attn_hbm.at[pl.ds(r0, RC)], abuf)
    pltpu.sync_copy(m_sh.at[pl.ds(m0, RC)], mbuf)

    def rbody(r, carry2):
      for k in range(HALF // LANE):
        sl = pl.ds(k * LANE, LANE)
        val = EPS * hbuf[r, sl] + abuf[r, sl] * mbuf[r, sl]
        hbuf[r, sl] = jnp.maximum(val, 0.0)
      return carry2
    lax.fori_loop(0, RC, rbody, 0)

    pltpu.sync_copy(hbuf, out_hbm.at[pl.ds(r0, RC)])
    return carry
  lax.fori_loop(0, nch, ep_body, 0)


@functools.cache
def _agg_kernel():
  mesh = plsc.VectorSubcoreMesh(**_MESH)
  return pl.kernel(
      _agg_body,
      out_type=jax.ShapeDtypeStruct((HS_H, HALF), jnp.float32),
      mesh=mesh,
      scratch_types=[
          pltpu.VMEM((EB,), jnp.int32),
          pltpu.VMEM((EB,), jnp.int32),
          pltpu.VMEM((EB,), jnp.int32),
          pltpu.VMEM((EB,), jnp.float32),
          pltpu.VMEM((EB, HALF), jnp.float32),
          pltpu.VMEM((RC, HALF), jnp.float32),
          pltpu.VMEM((RC, HALF), jnp.float32),
          pltpu.VMEM((RC, HALF), jnp.float32),
          pltpu.VMEM_SHARED((M_H, HALF), jnp.float32),
      ],
      **_SC_PARAMS,
  )


def _tca_body(ctx_ref, wc_ref, bc_ref, h_ref, wg_ref, deg_ref,
              attn_ref, p_ref, d_ref):
  attn_ref[...] = jnp.tanh(
      jnp.dot(ctx_ref[...], wc_ref[...], preferred_element_type=jnp.float32)
      + bc_ref[...])
  p_ref[...] = jnp.dot(h_ref[...], wg_ref[...],
                       preferred_element_type=jnp.float32)
  deg = deg_ref[0] + deg_ref[1]
  d_ref[...] = lax.rsqrt(jnp.clip(deg, 1.0, None))


@functools.cache
def _tca_kernel():
  grid = (N // R_TC,)
  return pl.pallas_call(
      _tca_body,
      grid=grid,
      in_specs=[
          pl.BlockSpec((R_TC, D), lambda i: (i, 0)),
          pl.BlockSpec((D, D), lambda i: (0, 0)),
          pl.BlockSpec((1, D), lambda i: (0, 0)),
          pl.BlockSpec((R_TC, D), lambda i: (i, 0)),
          pl.BlockSpec((D, HALF), lambda i: (0, 0)),
          pl.BlockSpec((NC, 8, HALF), lambda i: (0, i, 0)),
      ],
      out_specs=[
          pl.BlockSpec((R_TC, D), lambda i: (i, 0)),
          pl.BlockSpec((R_TC, HALF), lambda i: (i, 0)),
          pl.BlockSpec((8, HALF), lambda i: (i, 0)),
      ],
      out_shape=[
          jax.ShapeDtypeStruct((N, D), jnp.float32),
          jax.ShapeDtypeStruct((N, HALF), jnp.float32),
          jax.ShapeDtypeStruct((M_H // HALF, HALF), jnp.float32),
      ],
  )


def _tcb_body(h0_ref, h1_ref, wga_ref, wgb_ref, p_ref):
  p_ref[...] = (
      jnp.dot(h0_ref[...], wga_ref[...], preferred_element_type=jnp.float32)
      + jnp.dot(h1_ref[...], wgb_ref[...], preferred_element_type=jnp.float32))


@functools.cache
def _tcb_kernel():
  grid = (N // R_TC,)
  return pl.pallas_call(
      _tcb_body,
      grid=grid,
      in_specs=[
          pl.BlockSpec((R_TC, HALF), lambda i: (i, 0)),
          pl.BlockSpec((R_TC, HALF), lambda i: (i + N // R_TC, 0)),
          pl.BlockSpec((HALF, HALF), lambda i: (0, 0)),
          pl.BlockSpec((HALF, HALF), lambda i: (0, 0)),
      ],
      out_specs=pl.BlockSpec((R_TC, HALF), lambda i: (i, 0)),
      out_shape=jax.ShapeDtypeStruct((N, HALF), jnp.float32),
  )


def _slab(x):
  pad = jnp.zeros((HS_H - 2 * N, HALF), jnp.float32)
  return jnp.concatenate([x[:, :HALF], x[:, HALF:], pad], axis=0)


def _pad_table(x):
  return jnp.concatenate([x, jnp.zeros((NPAD - N,), jnp.float32)])


def kernel(h, context, edge_index, Wc, bc, gate_W, gate_b):
  src = edge_index[0]
  dst = edge_index[1]
  padfill = jnp.full((EPAD - E,), N, dtype=jnp.int32)
  src3 = jnp.concatenate([src, padfill]).reshape(NS, NB, EB)
  dst3 = jnp.concatenate([dst, padfill]).reshape(NS, NB, EB)
  src_f = src3.reshape(-1)
  dst_f = dst3.reshape(-1)

  deg2 = _deg_kernel()(dst3)
  deg_r = deg2.reshape(NC, M_H // HALF, HALF)

  wg0 = jnp.concatenate(
      [gate_W[0, :D], gate_W[0, D:], jnp.zeros((D, HALF - 2), jnp.float32)],
      axis=1)
  attn, p0, d2 = _tca_kernel()(context, Wc, bc.reshape(1, D), h, wg0, deg_r)

  d_p = _pad_table(d2.reshape(-1)[:N])
  attn_s = _slab(attn)
  hs = _slab(h)

  for i in range(L):
    if i == 0:
      p = p0
    else:
      wga = jnp.concatenate(
          [gate_W[1, :HALF], gate_W[1, D:D + HALF],
           jnp.zeros((HALF, HALF - 2), jnp.float32)], axis=1)
      wgb = jnp.concatenate(
          [gate_W[1, HALF:D], gate_W[1, D + HALF:],
           jnp.zeros((HALF, HALF - 2), jnp.float32)], axis=1)
      p = _tcb_kernel()(hs, hs, wga, wgb)
    u_p = _pad_table(p[:, 0] + gate_b[i, 0])
    v_p = _pad_table(p[:, 1])
    e3 = _gate_kernel()(src3, dst3, u_p, v_p, d_p)
    hs = _agg_kernel()(hs, attn_s, src_f, dst_f, e3.reshape(-1))

  return jnp.concatenate([hs[:N], hs[N:2 * N]], axis=1)
